# Initial kernel scaffold; baseline (speedup 1.0000x reference)
import jax
import jax.numpy as jnp
from jax.experimental import pallas as pl


def kernel(hidden_states, gate_weight, w_up, w_down):
    raise NotImplementedError("write your pallas kernel here")


# speedup vs baseline: 1.3907x; 1.3907x over previous
import functools

import jax
import jax.numpy as jnp
from jax.experimental import pallas as pl
from jax.experimental.pallas import tpu as pltpu

_NUM_EXPERTS = 8
_TOP_K = 2
_TOKEN_BLOCK = 1024


def _moe_body(gw_ref, x_ref, wu_ref, wd_ref, out_ref, gates_ref):
    e = pl.program_id(1)
    xb = x_ref[...]

    @pl.when(e == 0)
    def _router():
        raw = jax.lax.dot_general(
            xb, gw_ref[...],
            dimension_numbers=(((1,), (1,)), ((), ())),
            preferred_element_type=jnp.float32)
        lsig = -jax.nn.softplus(-raw)
        z = lsig - jnp.max(lsig, axis=-1, keepdims=True)
        ez = jnp.exp(z)
        probs = ez / jnp.sum(ez, axis=-1, keepdims=True)
        i1 = jnp.argmax(probs, axis=-1, keepdims=True)
        cols = jax.lax.broadcasted_iota(jnp.int32, probs.shape, 1)
        m1 = jnp.max(probs, axis=-1, keepdims=True)
        masked = jnp.where(cols == i1, -jnp.inf, probs)
        m2 = jnp.max(masked, axis=-1, keepdims=True)
        i2 = jnp.argmax(masked, axis=-1, keepdims=True)
        denom = m1 + m2 + 1e-20
        keep = (cols == i1) | (cols == i2)
        gates_ref[...] = jnp.where(keep, probs, 0.0) / denom

    up = jnp.dot(xb, wu_ref[0], preferred_element_type=jnp.float32)
    t = jnp.maximum(up, 0.0)
    act = (t * t).astype(jnp.bfloat16)
    down = jnp.dot(act, wd_ref[0], preferred_element_type=jnp.float32)
    gates = gates_ref[...]
    ecols = jax.lax.broadcasted_iota(jnp.int32, gates.shape, 1)
    g = jnp.sum(jnp.where(ecols == e, gates, 0.0), axis=1, keepdims=True)
    contrib = g * down

    @pl.when(e == 0)
    def _init():
        out_ref[...] = contrib

    @pl.when(e != 0)
    def _acc():
        out_ref[...] += contrib


@functools.partial(jax.jit, static_argnames=())
def kernel(hidden_states, gate_weight, w_up, w_down):
    B, S, H = hidden_states.shape
    T = B * S
    E = _NUM_EXPERTS
    I = w_up.shape[-1]
    x = hidden_states.reshape(T, H).astype(jnp.bfloat16)
    gw = gate_weight.astype(jnp.bfloat16)
    wu = w_up.astype(jnp.bfloat16)
    wd = w_down.astype(jnp.bfloat16)

    tb = _TOKEN_BLOCK
    grid = (T // tb, E)
    out = pl.pallas_call(
        _moe_body,
        grid=grid,
        in_specs=[
            pl.BlockSpec((E, H), lambda t, e: (0, 0)),
            pl.BlockSpec((tb, H), lambda t, e: (t, 0)),
            pl.BlockSpec((1, H, I), lambda t, e: (e, 0, 0)),
            pl.BlockSpec((1, I, H), lambda t, e: (e, 0, 0)),
        ],
        out_specs=pl.BlockSpec((tb, H), lambda t, e: (t, 0)),
        out_shape=jax.ShapeDtypeStruct((T, H), jnp.float32),
        scratch_shapes=[pltpu.VMEM((tb, E), jnp.float32)],
        compiler_params=pltpu.CompilerParams(
            dimension_semantics=("arbitrary", "arbitrary")),
    )(gw, x, wu, wd)
    return out.reshape(B, S, H)
